# baseline (speedup 1.0000x reference)
import jax
import jax.numpy as jnp
from jax import lax
from jax.experimental import pallas as pl
from jax.experimental.pallas import tpu as pltpu

_EMBED_DIM = 32
_NUM_SLOTS = 5
_POOL_OFF = 216


def _dot(a, b):
    return jnp.dot(a, b, preferred_element_type=jnp.float32)


def _fused_body(jokt_smem, scalars_ref, sel_ref, hand_ref, ht_ref, deck_ref,
                tablet_hbm, w1_ref, b1_ref, w2_ref, b2_ref, w3t_ref, b3_ref,
                out_ref, blocks_v, sem):
    ids = [jokt_smem[0, i].astype(jnp.int32) for i in range(_NUM_SLOTS)]
    copies = [
        pltpu.make_async_copy(
            tablet_hbm.at[:, pl.ds(pl.multiple_of(
                (ids[i] // 128) * 128, 128), 128)],
            blocks_v.at[i],
            sem,
        )
        for i in range(_NUM_SLOTS)
    ]
    for cp in copies:
        cp.start()

    h = _dot(scalars_ref[...], w1_ref[0:16, :])
    h = h + _dot(sel_ref[...], w1_ref[16:24, :])
    for r in range(8):
        h = h + _dot(hand_ref[pl.ds(r, 1), :], w1_ref[24 + 16 * r:40 + 16 * r, :])
    htdeck = jnp.concatenate([ht_ref[...], deck_ref[...]], axis=1)
    h = h + _dot(htdeck, w1_ref[152:216, :])
    lane8 = lax.broadcasted_iota(jnp.int32, (1, 8), 1)
    enabled = jnp.zeros((1, 8), jnp.float32)
    for i in range(_NUM_SLOTS):
        enabled = enabled + jnp.where(lane8 == i, jokt_smem[1, i],
                                      jnp.float32(0.0))
    h = h + _dot(enabled[:, 0:_NUM_SLOTS], w1_ref[248:253, :])

    cnt = jnp.float32(0.0)
    for i in range(_NUM_SLOTS):
        cnt = cnt + jnp.where(ids[i] > 0, jnp.float32(1.0), jnp.float32(0.0))
    inv = 1.0 / jnp.maximum(cnt, 1.0)
    lane = lax.broadcasted_iota(jnp.int32, (1, 128), 1)
    for cp in copies:
        cp.wait()
    acc = jnp.zeros((_EMBED_DIM, 1), jnp.float32)
    for i in range(_NUM_SLOTS):
        onehot = (lane == (ids[i] % 128)).astype(jnp.float32)
        acc = acc + jnp.sum(blocks_v[i] * onehot, axis=1, keepdims=True)
    pooled_col = acc * inv
    h = h + lax.dot_general(
        pooled_col, w1_ref[_POOL_OFF:_POOL_OFF + _EMBED_DIM, :],
        (((0,), (0,)), ((), ())), preferred_element_type=jnp.float32)

    h = jnp.maximum(h + b1_ref[...], 0.0)
    h = jnp.maximum(_dot(h, w2_ref[...]) + b2_ref[...], 0.0)
    out_ref[...] = lax.dot_general(
        h, w3t_ref[...], (((1,), (1,)), ((), ())),
        preferred_element_type=jnp.float32) + b3_ref[...]


def kernel(scalars, selection_mask, hand, hand_type, deck, jokers, emb_table,
           W1, b1, W2, b2, W3, b3):
    vmem = pl.BlockSpec(memory_space=pltpu.MemorySpace.VMEM)
    out = pl.pallas_call(
        _fused_body,
        in_specs=[
            pl.BlockSpec(memory_space=pltpu.MemorySpace.SMEM),
            vmem, vmem, vmem, vmem, vmem,
            pl.BlockSpec(memory_space=pltpu.MemorySpace.HBM),
            vmem, vmem, vmem, vmem, vmem, vmem,
        ],
        out_shape=jax.ShapeDtypeStruct((1, W3.shape[1]), jnp.float32),
        compiler_params=pltpu.CompilerParams(
            disable_bounds_checks=True, disable_semaphore_checks=True),
        scratch_shapes=[
            pltpu.VMEM((_NUM_SLOTS, _EMBED_DIM, 128), jnp.float32),
            pltpu.SemaphoreType.DMA,
        ],
    )(jokers.T, scalars.reshape(1, -1), selection_mask.reshape(1, -1), hand,
      hand_type.reshape(1, -1), deck.reshape(1, -1), emb_table.T, W1,
      b1.reshape(1, -1), W2, b2.reshape(1, -1), W3.T, b3.reshape(1, -1))
    return out.reshape(-1)
